# Initial kernel scaffold; baseline (speedup 1.0000x reference)
import jax
import jax.numpy as jnp
from jax.experimental import pallas as pl


def kernel(x, token_types, symbol_table, token_type_table, gamma, beta):
    raise NotImplementedError("write your pallas kernel here")


# speedup vs baseline: 3.3298x; 3.3298x over previous
import functools

import jax
import jax.numpy as jnp
from jax import lax
from jax.experimental import pallas as pl
from jax.experimental.pallas import tpu as pltpu
from jax.experimental.pallas import tpu_sc as plsc

EPS = 1e-12
NC = 2
NS = 16
NW = NC * NS
L = 16
CHUNK = 128


_GDN = lax.GatherDimensionNumbers(
    offset_dims=(), collapsed_slice_dims=(0,), start_index_map=(0,)
)


def _permute(v, p):
    return lax.gather(
        v, p[:, None], _GDN, slice_sizes=(1,),
        mode=lax.GatherScatterMode.PROMISE_IN_BOUNDS,
    )


def _xlane_sum(v, perms):
    for p in perms:
        v = v + _permute(v, p)
    return v


def _rsqrt(v):
    vi = lax.bitcast_convert_type(v, jnp.int32)
    yi = jnp.full((L,), 0x5F3759DF, jnp.int32) - lax.shift_right_arithmetic(vi, 1)
    y = lax.bitcast_convert_type(yi, jnp.float32)
    for _ in range(3):
        y = y * (1.5 - 0.5 * v * y * y)
    return y


def _ln_body(i, symrows, typrows, outrows, gs, bs, perms, D):
    nj = D // L
    es = []
    for j in range(nj):
        s = symrows[i, pl.ds(j * L, L)]
        t = typrows[i, pl.ds(j * L, L)]
        es.append(s + t)
    acc = es[0]
    for j in range(1, nj):
        acc = acc + es[j]
    acc2 = es[0] * es[0]
    for j in range(1, nj):
        acc2 = acc2 + es[j] * es[j]
    mean = _xlane_sum(acc, perms) * (1.0 / D)
    meansq = _xlane_sum(acc2, perms) * (1.0 / D)
    var = meansq - mean * mean
    rstd = _rsqrt(var + EPS)
    mrs = mean * rstd
    for j in range(nj):
        a = gs[j] * rstd
        c = bs[j] - gs[j] * mrs
        outrows[i, pl.ds(j * L, L)] = es[j] * a + c


def _sc_kernel(x_hbm, tt_hbm, sym_hbm, typ_hbm, g_hbm, b_hbm, out_hbm,
               xidx, ttidx, symrows, typrows, outrows, g_v, b_v,
               sem0, sem1, *, per_w, D):
    wid = lax.axis_index("s") * NC + lax.axis_index("c")
    pltpu.sync_copy(g_hbm, g_v)
    pltpu.sync_copy(b_hbm, b_v)
    nj = D // L
    gs = tuple(g_v[pl.ds(j * L, L)] for j in range(nj))
    bs = tuple(b_v[pl.ds(j * L, L)] for j in range(nj))
    lane = lax.iota(jnp.int32, L)
    perms = tuple(jnp.bitwise_xor(lane, k) for k in (8, 4, 2, 1))
    nchunks = per_w // CHUNK

    def chunk_body(c, carry):
        base = wid * per_w + c * CHUNK
        pltpu.sync_copy(x_hbm.at[pl.ds(base, CHUNK)], xidx)
        pltpu.sync_copy(tt_hbm.at[pl.ds(base, CHUNK)], ttidx)
        cp0 = pltpu.async_copy(sym_hbm.at[xidx], symrows, sem0)
        cp1 = pltpu.async_copy(typ_hbm.at[ttidx], typrows, sem1)
        cp0.wait()
        cp1.wait()

        def tok_body(i, tc):
            _ln_body(i, symrows, typrows, outrows, gs, bs, perms, D)
            return tc

        lax.fori_loop(0, CHUNK, tok_body, 0)
        pltpu.sync_copy(outrows, out_hbm.at[pl.ds(base, CHUNK)])
        return carry

    lax.fori_loop(0, nchunks, chunk_body, 0)


def kernel(x, token_types, symbol_table, token_type_table, gamma, beta):
    B, S = x.shape
    V, D = symbol_table.shape
    N = B * S
    assert N % (NW * CHUNK) == 0
    per_w = N // NW

    xf = x.reshape(N).astype(jnp.int32)
    tf = token_types.reshape(N).astype(jnp.int32)

    mesh = plsc.VectorSubcoreMesh(
        core_axis_name="c", subcore_axis_name="s", num_cores=NC, num_subcores=NS
    )
    run = pl.kernel(
        functools.partial(_sc_kernel, per_w=per_w, D=D),
        out_type=jax.ShapeDtypeStruct((N, D), jnp.float32),
        mesh=mesh,
        scratch_types=[
            pltpu.VMEM((CHUNK,), jnp.int32),
            pltpu.VMEM((CHUNK,), jnp.int32),
            pltpu.VMEM((CHUNK, D), jnp.float32),
            pltpu.VMEM((CHUNK, D), jnp.float32),
            pltpu.VMEM((CHUNK, D), jnp.float32),
            pltpu.VMEM((D,), jnp.float32),
            pltpu.VMEM((D,), jnp.float32),
            pltpu.SemaphoreType.DMA,
            pltpu.SemaphoreType.DMA,
        ],
    )
    out = run(xf, tf, symbol_table, token_type_table, gamma, beta)
    return out.reshape(B, S, D)


# speedup vs baseline: 3.4008x; 1.0213x over previous
import functools

import jax
import jax.numpy as jnp
from jax import lax
from jax.experimental import pallas as pl
from jax.experimental.pallas import tpu as pltpu
from jax.experimental.pallas import tpu_sc as plsc

EPS = 1e-12
NC = 2
NS = 16
NW = NC * NS
L = 16
CHUNK = 64

_GDN = lax.GatherDimensionNumbers(
    offset_dims=(), collapsed_slice_dims=(0,), start_index_map=(0,)
)


def _permute(v, p):
    return lax.gather(
        v, p[:, None], _GDN, slice_sizes=(1,),
        mode=lax.GatherScatterMode.PROMISE_IN_BOUNDS,
    )


def _xlane_sum(v, perms):
    for p in perms:
        v = v + _permute(v, p)
    return v


def _rsqrt(v):
    vi = lax.bitcast_convert_type(v, jnp.int32)
    yi = jnp.full((L,), 0x5F3759DF, jnp.int32) - lax.shift_right_arithmetic(vi, 1)
    y = lax.bitcast_convert_type(yi, jnp.float32)
    for _ in range(3):
        y = y * (1.5 - 0.5 * v * y * y)
    return y


def _ln_body(i, symrows, typrows, outrows, gs, bs, perms, D):
    nj = D // L
    es = []
    for j in range(nj):
        s = symrows[i, pl.ds(j * L, L)]
        t = typrows[i, pl.ds(j * L, L)]
        es.append(s + t)
    acc = es[0]
    for j in range(1, nj):
        acc = acc + es[j]
    acc2 = es[0] * es[0]
    for j in range(1, nj):
        acc2 = acc2 + es[j] * es[j]
    mean = _xlane_sum(acc, perms) * (1.0 / D)
    meansq = _xlane_sum(acc2, perms) * (1.0 / D)
    var = meansq - mean * mean
    rstd = _rsqrt(var + EPS)
    mrs = mean * rstd
    for j in range(nj):
        a = gs[j] * rstd
        c = bs[j] - gs[j] * mrs
        outrows[i, pl.ds(j * L, L)] = es[j] * a + c


def _sc_kernel(x_hbm, tt_hbm, sym_hbm, typ_hbm, g_hbm, b_hbm, out_hbm,
               xidx, ttidx, symrows, typrows, outrows, g_v, b_v,
               gs0, gs1, gt0, gt1, os0, os1, *, per_w, D):
    wid = lax.axis_index("s") * NC + lax.axis_index("c")
    w0 = wid * per_w
    pltpu.sync_copy(g_hbm, g_v)
    pltpu.sync_copy(b_hbm, b_v)
    pltpu.sync_copy(x_hbm.at[pl.ds(w0, per_w)], xidx)
    pltpu.sync_copy(tt_hbm.at[pl.ds(w0, per_w)], ttidx)
    nj = D // L
    gs = tuple(g_v[pl.ds(j * L, L)] for j in range(nj))
    bs = tuple(b_v[pl.ds(j * L, L)] for j in range(nj))
    lane = lax.iota(jnp.int32, L)
    perms = tuple(jnp.bitwise_xor(lane, k) for k in (8, 4, 2, 1))
    n = per_w // CHUNK
    gsems = (gs0, gs1)
    tsems = (gt0, gt1)
    osems = (os0, os1)

    def _gathers(c, b):
        ix = xidx.at[pl.ds(c * CHUNK, CHUNK)]
        it = ttidx.at[pl.ds(c * CHUNK, CHUNK)]
        cps = pltpu.make_async_copy(sym_hbm.at[ix], symrows.at[b], gsems[b])
        cpt = pltpu.make_async_copy(typ_hbm.at[it], typrows.at[b], tsems[b])
        return cps, cpt

    def _outcopy(c, b):
        dst = out_hbm.at[pl.ds(w0 + c * CHUNK, CHUNK)]
        return pltpu.make_async_copy(outrows.at[b], dst, osems[b])

    for b in range(2):
        cps, cpt = _gathers(b, b)
        cps.start()
        cpt.start()

    def pair_body(k, carry):
        for b in range(2):
            c = 2 * k + b
            cps, cpt = _gathers(c, b)
            cps.wait()
            cpt.wait()

            @pl.when(c >= 2)
            def _():
                _outcopy(c - 2, b).wait()

            sr, tr, orr = symrows.at[b], typrows.at[b], outrows.at[b]

            @plsc.parallel_loop(0, CHUNK, 1, unroll=4)
            def _token(i):
                _ln_body(i, sr, tr, orr, gs, bs, perms, D)

            _outcopy(c, b).start()

            @pl.when(c + 2 < n)
            def _():
                cps2, cpt2 = _gathers(c + 2, b)
                cps2.start()
                cpt2.start()
        return carry

    lax.fori_loop(0, n // 2, pair_body, 0)
    for b in range(2):
        _outcopy(n - 2 + b, b).wait()


def kernel(x, token_types, symbol_table, token_type_table, gamma, beta):
    B, S = x.shape
    V, D = symbol_table.shape
    N = B * S
    assert N % (NW * 2 * CHUNK) == 0
    per_w = N // NW

    xf = x.reshape(N).astype(jnp.int32)
    tf = token_types.reshape(N).astype(jnp.int32)

    mesh = plsc.VectorSubcoreMesh(
        core_axis_name="c", subcore_axis_name="s", num_cores=NC, num_subcores=NS
    )
    run = pl.kernel(
        functools.partial(_sc_kernel, per_w=per_w, D=D),
        out_type=jax.ShapeDtypeStruct((N, D), jnp.float32),
        mesh=mesh,
        scratch_types=[
            pltpu.VMEM((N // NW,), jnp.int32),
            pltpu.VMEM((N // NW,), jnp.int32),
            pltpu.VMEM((2, CHUNK, D), jnp.float32),
            pltpu.VMEM((2, CHUNK, D), jnp.float32),
            pltpu.VMEM((2, CHUNK, D), jnp.float32),
            pltpu.VMEM((D,), jnp.float32),
            pltpu.VMEM((D,), jnp.float32),
            pltpu.SemaphoreType.DMA,
            pltpu.SemaphoreType.DMA,
            pltpu.SemaphoreType.DMA,
            pltpu.SemaphoreType.DMA,
            pltpu.SemaphoreType.DMA,
            pltpu.SemaphoreType.DMA,
        ],
    )
    out = run(xf, tf, symbol_table, token_type_table, gamma, beta)
    return out.reshape(B, S, D)
